# baseline (speedup 1.0000x reference)
import functools

import jax
import jax.numpy as jnp
from jax import lax
from jax.experimental import pallas as pl
from jax.experimental.pallas import tpu as pltpu
from jax.experimental.pallas import tpu_sc as plsc

ALPHA = 0.15
K = 10
N = 25000
NPAD = 25088
NROW = NPAD // 16
NSUB = 28
SUB = NROW // NSUB
D = 64
E = 400000
CB = 128
NCHUNK = 196
EPT = NCHUNK * CB
ET = EPT * 16
B = 16384

_f32 = jnp.float32
_mesh = plsc.VectorSubcoreMesh(core_axis_name="c", subcore_axis_name="s")
_sc_params = pltpu.CompilerParams(use_tc_tiling_on_sc=False)


def _fill(ref, value):
    v = jnp.full((16,), value, _f32)
    nrows = ref.shape[0]

    def body(r, carry):
        for c in range(ref.shape[1] // 16):
            ref[r, pl.ds(c * 16, 16)] = v
        return carry

    lax.fori_loop(0, nrows, body, 0)


@functools.partial(
    pl.kernel,
    out_type=(jax.ShapeDtypeStruct((NPAD, D), _f32),
              jax.ShapeDtypeStruct((NPAD, D), _f32)),
    mesh=_mesh,
    scratch_types=[
        pltpu.VMEM((CB,), jnp.int32),
        pltpu.VMEM((CB, D), _f32),
        pltpu.VMEM((SUB, D), _f32),
        pltpu.VMEM((SUB, D), _f32),
        pltpu.VMEM_SHARED((NPAD, D), _f32),
    ],
    compiler_params=_sc_params,
)
def _sc_counts(dui, diu, reci_out, recu_out, didx, ones, ba, bb, acc):
    cid = lax.axis_index("c")
    sid = lax.axis_index("s")

    _fill(ba, 0.0)
    _fill(ones, 1.0)
    base = sid * NROW

    def zbody(k, carry):
        pltpu.sync_copy(ba, acc.at[pl.ds(base + k * SUB, SUB)])
        return carry
    lax.fori_loop(0, NSUB, zbody, 0)
    plsc.subcore_barrier()

    def count(dm):
        def body(j, carry):
            row = sid * NCHUNK + j
            pltpu.sync_copy(dm.at[row], didx)
            pltpu.sync_copy(ones, acc.at[didx], add=True)
            return carry
        lax.fori_loop(0, NCHUNK, body, 0)

    @pl.when(cid == 0)
    def _():
        count(dui)

    @pl.when(cid == 1)
    def _():
        count(diu)

    plsc.subcore_barrier()

    def emit(out):
        def sub(k, carry):
            rb = base + k * SUB
            pltpu.sync_copy(acc.at[pl.ds(rb, SUB)], ba)

            def body(r, c2):
                for c in range(4):
                    s = pl.ds(c * 16, 16)
                    bb[r, s] = (1.0 - ALPHA) / jnp.maximum(ba[r, s], 1.0)
                return c2
            lax.fori_loop(0, SUB, body, 0)
            pltpu.sync_copy(bb, out.at[pl.ds(rb, SUB)])
            return carry
        lax.fori_loop(0, NSUB, sub, 0)

    @pl.when(cid == 0)
    def _():
        emit(reci_out)

    @pl.when(cid == 1)
    def _():
        emit(recu_out)


GRP = 14
NGRP = NCHUNK // GRP
NBUF = 2


@functools.partial(
    pl.kernel,
    out_type=(jax.ShapeDtypeStruct((NPAD, D), _f32),
              jax.ShapeDtypeStruct((NPAD, D), _f32)),
    mesh=_mesh,
    scratch_types=[
        pltpu.VMEM((GRP, CB), jnp.int32),
        pltpu.VMEM((GRP, CB), jnp.int32),
        pltpu.VMEM((CB, D), _f32),
        pltpu.VMEM((CB, D), _f32),
        pltpu.VMEM((CB, D), _f32),
        pltpu.SemaphoreType.DMA,
        pltpu.SemaphoreType.DMA,
        pltpu.SemaphoreType.DMA,
        pltpu.SemaphoreType.DMA,
        pltpu.SemaphoreType.DMA,
        pltpu.SemaphoreType.DMA,
        pltpu.VMEM_SHARED((NPAD, D), _f32),
    ],
    compiler_params=_sc_params,
)
def _sc_prop(hu, hi, h0u, h0i, recu, reci, sui, dui, siu, diu,
             hu_out, hi_out, sgrp, dgrp, rowsA, rowsB, rowsC,
             gs0, gs1, gs2, ss0, ss1, ss2, acc):
    cid = lax.axis_index("c")
    sid = lax.axis_index("s")
    rows = (rowsA, rowsB, rowsC)
    gsems = (gs0, gs1, gs2)
    ssems = (ss0, ss1, ss2)
    base = sid * NROW

    _fill(rowsA, 0.0)
    zd = []
    for k in range(NROW // CB):
        zd.append(pltpu.async_copy(rowsA, acc.at[pl.ds(base + k * CB, CB)],
                                   gsems[k % 2]))
    zd.append(pltpu.async_copy(rowsA.at[pl.ds(0, NROW % CB)],
                               acc.at[pl.ds(base + (NROW // CB) * CB, NROW % CB)],
                               gsems[2]))
    for d in zd:
        d.wait()
    plsc.subcore_barrier()

    def edges(h, sm, dm):
        def group(g, carry):
            brow = sid * NCHUNK + g * GRP
            pltpu.sync_copy(sm.at[pl.ds(brow, GRP)], sgrp)
            pltpu.sync_copy(dm.at[pl.ds(brow, GRP)], dgrp)
            gd, sd = {}, {}
            for t in range(min(NBUF, GRP)):
                gd[t] = pltpu.async_copy(h.at[sgrp.at[t]], rows[t % NBUF],
                                         gsems[t % NBUF])
            for t in range(GRP):
                b = t % NBUF
                gd[t].wait()
                sd[t] = pltpu.async_copy(rows[b], acc.at[dgrp.at[t]],
                                         ssems[b], add=True)
                u = t + NBUF
                if u < GRP:
                    sd[t].wait()
                    gd[u] = pltpu.async_copy(h.at[sgrp.at[u]], rows[b],
                                             gsems[b])
            for t in range(GRP - NBUF, GRP):
                sd[t].wait()
            return carry
        lax.fori_loop(0, NGRP, group, 0)

    @pl.when(cid == 0)
    def _():
        edges(hu, sui, dui)

    @pl.when(cid == 1)
    def _():
        edges(hi, siu, diu)

    plsc.subcore_barrier()

    def blend(rec, h0, hout):
        def sub_block(rb, nr):
            d0 = pltpu.async_copy(acc.at[pl.ds(rb, nr)], rowsA.at[pl.ds(0, nr)], gs0)
            d1 = pltpu.async_copy(rec.at[pl.ds(rb, nr)], rowsB.at[pl.ds(0, nr)], gs1)
            d2 = pltpu.async_copy(h0.at[pl.ds(rb, nr)], rowsC.at[pl.ds(0, nr)], gs2)
            d0.wait()
            d1.wait()
            d2.wait()

            def body(r, c2):
                for c in range(4):
                    s = pl.ds(c * 16, 16)
                    rowsC[r, s] = rowsA[r, s] * rowsB[r, s] + ALPHA * rowsC[r, s]
                return c2
            lax.fori_loop(0, nr, body, 0)
            pltpu.sync_copy(rowsC.at[pl.ds(0, nr)], hout.at[pl.ds(rb, nr)])

        def sub(k, carry):
            sub_block(base + k * CB, CB)
            return carry
        lax.fori_loop(0, NROW // CB, sub, 0)
        sub_block(base + (NROW // CB) * CB, NROW % CB)

    @pl.when(cid == 0)
    def _():
        blend(reci, h0i, hi_out)

    @pl.when(cid == 1)
    def _():
        blend(recu, h0u, hu_out)


def _make_gather(nch, nrows):
    @functools.partial(
        pl.kernel,
        out_type=jax.ShapeDtypeStruct((nrows, D), _f32),
        mesh=_mesh,
        scratch_types=[
            pltpu.VMEM((CB,), jnp.int32),
            pltpu.VMEM((CB, D), _f32),
            pltpu.SemaphoreType.DMA,
        ],
        compiler_params=_sc_params,
    )
    def g(table, idxm, out, sidx, rows, sem):
        cid = lax.axis_index("c")
        sid = lax.axis_index("s")
        wid = sid * 2 + cid

        def body(j, carry):
            row = wid * nch + j
            pltpu.sync_copy(idxm.at[row], sidx)
            pltpu.async_copy(table.at[sidx], rows, sem).wait()
            pltpu.sync_copy(rows, out.at[pl.ds(row * CB, CB)])
            return carry
        lax.fori_loop(0, nch, body, 0)

    return g


_gather_u = _make_gather(4, B)
_gather_pn = _make_gather(8, 2 * B)


_MB = 3136


def _ln(v, g, e):
    mu = jnp.mean(v, axis=-1, keepdims=True)
    var = jnp.mean((v - mu) ** 2, axis=-1, keepdims=True)
    return (v - mu) / jnp.sqrt(var + 1e-5) * g + e


def _dot(a, b):
    return jax.lax.dot_general(a, b, (((1,), (0,)), ((), ())),
                               precision=jax.lax.Precision.HIGHEST,
                               preferred_element_type=_f32)


def _elu(v):
    return jnp.where(v > 0, v, jnp.exp(jnp.minimum(v, 0.0)) - 1.0)


def _mlp_body(x_ref, w1, b1, g1, e1, w2, b2, g2, e2, w3, b3, o_ref):
    x = x_ref[...]
    h = _elu(_ln(_dot(x, w1[0]) + b1[0], g1[0], e1[0]))
    h = _elu(_ln(_dot(h, w2[0]) + b2[0], g2[0], e2[0]))
    o_ref[...] = _dot(h, w3[0]) + b3[0]


def _stkv(a, b):
    return jnp.stack([a, b]).reshape(2, 1, -1)


def _mlp_all(x_all, w1s, b1s, g1s, e1s, w2s, b2s, g2s, e2s, w3s, b3s):
    nblk = (2 * NPAD) // _MB
    half = nblk // 2
    wmap = lambda i: (i // half, 0, 0)
    vmap = lambda i: (i // half, 0)
    return pl.pallas_call(
        _mlp_body,
        grid=(nblk,),
        in_specs=[
            pl.BlockSpec((_MB, 128), lambda i: (i, 0)),
            pl.BlockSpec((1, 128, 128), wmap),
            pl.BlockSpec((1, 1, 128), wmap),
            pl.BlockSpec((1, 1, 128), wmap),
            pl.BlockSpec((1, 1, 128), wmap),
            pl.BlockSpec((1, 128, 128), wmap),
            pl.BlockSpec((1, 1, 128), wmap),
            pl.BlockSpec((1, 1, 128), wmap),
            pl.BlockSpec((1, 1, 128), wmap),
            pl.BlockSpec((1, 128, D), wmap),
            pl.BlockSpec((1, 1, D), wmap),
        ],
        out_specs=pl.BlockSpec((_MB, D), lambda i: (i, 0)),
        out_shape=jax.ShapeDtypeStruct((2 * NPAD, D), _f32),
    )(x_all, w1s, b1s, g1s, e1s, w2s, b2s, g2s, e2s, w3s, b3s)


_PB = 1024


def _pred_body(zu_ref, zp_ref, zn_ref, w1, b1, w2, b2, w3, o_ref):
    i = pl.program_id(0)
    zu = zu_ref[...]

    def head(z2):
        x = jnp.concatenate([zu, z2], axis=1)
        h = jnp.maximum(_dot(x, w1[...]) + b1[...], 0.0)
        h = jnp.maximum(_dot(h, w2[...]) + b2[...], 0.0)
        return _dot(h, w3[...])

    d = head(zp_ref[...]) - head(zn_ref[...])
    ls = jnp.minimum(d, 0.0) - jnp.log1p(jnp.exp(-jnp.abs(d)))

    @pl.when(i == 0)
    def _():
        o_ref[...] = jnp.zeros((1, 1), _f32)

    o_ref[...] += jnp.sum(ls).reshape(1, 1)


def _pred_loss(zu, zpn, pW1, pb1, pW2, pb2, pW3):
    nblk = B // _PB
    out = pl.pallas_call(
        _pred_body,
        grid=(nblk,),
        in_specs=[
            pl.BlockSpec((_PB, D), lambda i: (i, 0)),
            pl.BlockSpec((_PB, D), lambda i: (i, 0)),
            pl.BlockSpec((_PB, D), lambda i: (i + nblk, 0)),
            pl.BlockSpec((2 * D, 128), lambda i: (0, 0)),
            pl.BlockSpec((1, 128), lambda i: (0, 0)),
            pl.BlockSpec((128, D), lambda i: (0, 0)),
            pl.BlockSpec((1, D), lambda i: (0, 0)),
            pl.BlockSpec((D, 1), lambda i: (0, 0)),
        ],
        out_specs=pl.BlockSpec((1, 1), lambda i: (0, 0)),
        out_shape=jax.ShapeDtypeStruct((1, 1), _f32),
    )(zu, zpn, zpn, pW1, pb1.reshape(1, 128), pW2, pb2.reshape(1, D), pW3)
    return -out[0, 0] / B


def _prep_edges(ei):
    pad = ET - E
    src = jnp.concatenate([ei[0], jnp.zeros((pad,), jnp.int32)])
    dst = jnp.concatenate([ei[1], jnp.full((pad,), N, jnp.int32)])
    return src.reshape(-1, CB), dst.reshape(-1, CB)


def _pad_rows(x):
    return jnp.concatenate([x, jnp.zeros((NPAD - N, x.shape[1]), x.dtype)])


def kernel(x_user, x_item, edge_index_ui, edge_index_iu, user_idx, pos_idx, neg_idx,
           uW1, ub1, ug1, ue1, uW2, ub2, ug2, ue2, uW3, ub3,
           iW1, ib1, ig1, ie1, iW2, ib2, ig2, ie2, iW3, ib3,
           pW1, pb1, pW2, pb2, pW3, pb3):
    x_all = jnp.concatenate([_pad_rows(x_user), _pad_rows(x_item)])
    stk = lambda a, b: jnp.stack([a, b])
    h0 = _mlp_all(x_all,
                  stk(uW1, iW1), _stkv(ub1, ib1), _stkv(ug1, ig1), _stkv(ue1, ie1),
                  stk(uW2, iW2), _stkv(ub2, ib2), _stkv(ug2, ig2), _stkv(ue2, ie2),
                  stk(uW3, iW3), _stkv(ub3, ib3))
    h0u, h0i = h0[:NPAD], h0[NPAD:]

    sui, dui = _prep_edges(edge_index_ui)
    siu, diu = _prep_edges(edge_index_iu)

    reci, recu = _sc_counts(dui, diu)

    hu, hi = h0u, h0i
    for _ in range(K):
        hu, hi = _sc_prop(hu, hi, h0u, h0i, recu, reci, sui, dui, siu, diu)

    zu = _gather_u(hu, user_idx.reshape(-1, CB))
    zpn = _gather_pn(hi, jnp.concatenate([pos_idx, neg_idx]).reshape(-1, CB))

    return _pred_loss(zu, zpn, pW1, pb1, pW2, pb2, pW3)


# speedup vs baseline: 1.1303x; 1.1303x over previous
import functools

import jax
import jax.numpy as jnp
from jax import lax
from jax.experimental import pallas as pl
from jax.experimental.pallas import tpu as pltpu
from jax.experimental.pallas import tpu_sc as plsc

ALPHA = 0.15
K = 10
N = 25000
NPAD = 25088
NROW = NPAD // 16
D = 64
E = 400000
CB = 128
NCHUNK = 196
EPT = NCHUNK * CB
ET = EPT * 16
B = 16384

_f32 = jnp.float32
_mesh = plsc.VectorSubcoreMesh(core_axis_name="c", subcore_axis_name="s")
_sc_params = pltpu.CompilerParams(use_tc_tiling_on_sc=False)


def _fill(ref, value):
    v = jnp.full((16,), value, _f32)
    nrows = ref.shape[0]

    def body(r, carry):
        for c in range(ref.shape[1] // 16):
            ref[r, pl.ds(c * 16, 16)] = v
        return carry

    lax.fori_loop(0, nrows, body, 0)


GRP = 14
NGRP = NCHUNK // GRP
NBUF = 3


@functools.partial(
    pl.kernel,
    out_type=(jax.ShapeDtypeStruct((NPAD, D), _f32),
              jax.ShapeDtypeStruct((NPAD, D), _f32)),
    mesh=_mesh,
    scratch_types=[
        pltpu.VMEM((GRP, CB), jnp.int32),
        pltpu.VMEM((CB, D), _f32),
        pltpu.VMEM((CB, D), _f32),
        pltpu.VMEM((CB, D), _f32),
        pltpu.SemaphoreType.DMA,
        pltpu.SemaphoreType.DMA,
        pltpu.SemaphoreType.DMA,
        pltpu.VMEM_SHARED((NPAD, D), _f32),
    ],
    compiler_params=_sc_params,
)
def _sc_counts(dui, diu, reci_out, recu_out, dgrp, ones, ba, bb,
               ss0, ss1, ss2, acc):
    cid = lax.axis_index("c")
    sid = lax.axis_index("s")
    ssems = (ss0, ss1, ss2)
    base = sid * NROW

    _fill(ba, 0.0)
    _fill(ones, 1.0)
    zd = []
    for k in range(NROW // CB):
        zd.append(pltpu.async_copy(ba, acc.at[pl.ds(base + k * CB, CB)],
                                   ssems[k % 2]))
    zd.append(pltpu.async_copy(ba.at[pl.ds(0, NROW % CB)],
                               acc.at[pl.ds(base + (NROW // CB) * CB, NROW % CB)],
                               ssems[2]))
    for d in zd:
        d.wait()
    plsc.subcore_barrier()

    def count(dm):
        def group(g, carry):
            brow = sid * NCHUNK + g * GRP
            pltpu.sync_copy(dm.at[pl.ds(brow, GRP)], dgrp)
            sd = {}
            for t in range(GRP):
                b = t % NBUF
                if t >= NBUF:
                    sd[t - NBUF].wait()
                sd[t] = pltpu.async_copy(ones, acc.at[dgrp.at[t]],
                                         ssems[b], add=True)
            for t in range(GRP - NBUF, GRP):
                sd[t].wait()
            return carry
        lax.fori_loop(0, NGRP, group, 0)

    @pl.when(cid == 0)
    def _():
        count(dui)

    @pl.when(cid == 1)
    def _():
        count(diu)

    plsc.subcore_barrier()

    def emit(out):
        def sub_block(rb, nr):
            d0 = pltpu.async_copy(acc.at[pl.ds(rb, nr)], ba.at[pl.ds(0, nr)], ss0)
            d0.wait()

            def body(r, c2):
                for c in range(4):
                    s = pl.ds(c * 16, 16)
                    bb[r, s] = (1.0 - ALPHA) / jnp.maximum(ba[r, s], 1.0)
                return c2
            lax.fori_loop(0, nr, body, 0)
            pltpu.sync_copy(bb.at[pl.ds(0, nr)], out.at[pl.ds(rb, nr)])

        def sub(k, carry):
            sub_block(base + k * CB, CB)
            return carry
        lax.fori_loop(0, NROW // CB, sub, 0)
        sub_block(base + (NROW // CB) * CB, NROW % CB)

    @pl.when(cid == 0)
    def _():
        emit(reci_out)

    @pl.when(cid == 1)
    def _():
        emit(recu_out)


@functools.partial(
    pl.kernel,
    out_type=(jax.ShapeDtypeStruct((NPAD, D), _f32),
              jax.ShapeDtypeStruct((NPAD, D), _f32)),
    mesh=_mesh,
    scratch_types=[
        pltpu.VMEM((GRP, CB), jnp.int32),
        pltpu.VMEM((GRP, CB), jnp.int32),
        pltpu.VMEM((CB, D), _f32),
        pltpu.VMEM((CB, D), _f32),
        pltpu.VMEM((CB, D), _f32),
        pltpu.SemaphoreType.DMA,
        pltpu.SemaphoreType.DMA,
        pltpu.SemaphoreType.DMA,
        pltpu.SemaphoreType.DMA,
        pltpu.SemaphoreType.DMA,
        pltpu.SemaphoreType.DMA,
        pltpu.VMEM_SHARED((NPAD, D), _f32),
    ],
    compiler_params=_sc_params,
)
def _sc_prop(hu, hi, h0u, h0i, recu, reci, sui, dui, siu, diu,
             hu_out, hi_out, sgrp, dgrp, rowsA, rowsB, rowsC,
             gs0, gs1, gs2, ss0, ss1, ss2, acc):
    cid = lax.axis_index("c")
    sid = lax.axis_index("s")
    rows = (rowsA, rowsB, rowsC)
    gsems = (gs0, gs1, gs2)
    ssems = (ss0, ss1, ss2)
    base = sid * NROW

    _fill(rowsA, 0.0)
    zd = []
    for k in range(NROW // CB):
        zd.append(pltpu.async_copy(rowsA, acc.at[pl.ds(base + k * CB, CB)],
                                   gsems[k % 2]))
    zd.append(pltpu.async_copy(rowsA.at[pl.ds(0, NROW % CB)],
                               acc.at[pl.ds(base + (NROW // CB) * CB, NROW % CB)],
                               gsems[2]))
    for d in zd:
        d.wait()
    plsc.subcore_barrier()

    def edges(h, sm, dm):
        def group(g, carry):
            brow = sid * NCHUNK + g * GRP
            i0 = pltpu.async_copy(sm.at[pl.ds(brow, GRP)], sgrp, gs0)
            i1 = pltpu.async_copy(dm.at[pl.ds(brow, GRP)], dgrp, gs1)
            i0.wait()
            i1.wait()
            gd, sd = {}, {}
            for t in range(min(NBUF, GRP)):
                gd[t] = pltpu.async_copy(h.at[sgrp.at[t]], rows[t % NBUF],
                                         gsems[t % NBUF])
            for t in range(GRP):
                b = t % NBUF
                gd[t].wait()
                sd[t] = pltpu.async_copy(rows[b], acc.at[dgrp.at[t]],
                                         ssems[b], add=True)
                u = t + NBUF
                if u < GRP:
                    sd[t].wait()
                    gd[u] = pltpu.async_copy(h.at[sgrp.at[u]], rows[b],
                                             gsems[b])
            for t in range(GRP - NBUF, GRP):
                sd[t].wait()
            return carry
        lax.fori_loop(0, NGRP, group, 0)

    @pl.when(cid == 0)
    def _():
        edges(hu, sui, dui)

    @pl.when(cid == 1)
    def _():
        edges(hi, siu, diu)

    plsc.subcore_barrier()

    def blend(rec, h0, hout):
        def sub_block(rb, nr):
            d0 = pltpu.async_copy(acc.at[pl.ds(rb, nr)], rowsA.at[pl.ds(0, nr)], gs0)
            d1 = pltpu.async_copy(rec.at[pl.ds(rb, nr)], rowsB.at[pl.ds(0, nr)], gs1)
            d2 = pltpu.async_copy(h0.at[pl.ds(rb, nr)], rowsC.at[pl.ds(0, nr)], gs2)
            d0.wait()
            d1.wait()
            d2.wait()

            def body(r, c2):
                for c in range(4):
                    s = pl.ds(c * 16, 16)
                    rowsC[r, s] = rowsA[r, s] * rowsB[r, s] + ALPHA * rowsC[r, s]
                return c2
            lax.fori_loop(0, nr, body, 0)
            pltpu.sync_copy(rowsC.at[pl.ds(0, nr)], hout.at[pl.ds(rb, nr)])

        def sub(k, carry):
            sub_block(base + k * CB, CB)
            return carry
        lax.fori_loop(0, NROW // CB, sub, 0)
        sub_block(base + (NROW // CB) * CB, NROW % CB)

    @pl.when(cid == 0)
    def _():
        blend(reci, h0i, hi_out)

    @pl.when(cid == 1)
    def _():
        blend(recu, h0u, hu_out)


def _make_gather(nch, nrows):
    @functools.partial(
        pl.kernel,
        out_type=jax.ShapeDtypeStruct((nrows, D), _f32),
        mesh=_mesh,
        scratch_types=[
            pltpu.VMEM((CB,), jnp.int32),
            pltpu.VMEM((CB, D), _f32),
            pltpu.SemaphoreType.DMA,
        ],
        compiler_params=_sc_params,
    )
    def g(table, idxm, out, sidx, rows, sem):
        cid = lax.axis_index("c")
        sid = lax.axis_index("s")
        wid = sid * 2 + cid

        def body(j, carry):
            row = wid * nch + j
            pltpu.sync_copy(idxm.at[row], sidx)
            pltpu.async_copy(table.at[sidx], rows, sem).wait()
            pltpu.sync_copy(rows, out.at[pl.ds(row * CB, CB)])
            return carry
        lax.fori_loop(0, nch, body, 0)

    return g


_gather_u = _make_gather(4, B)
_gather_pn = _make_gather(8, 2 * B)


_MB = 3136


def _ln(v, g, e):
    mu = jnp.mean(v, axis=-1, keepdims=True)
    var = jnp.mean((v - mu) ** 2, axis=-1, keepdims=True)
    return (v - mu) / jnp.sqrt(var + 1e-5) * g + e


def _dot(a, b):
    return jax.lax.dot_general(a, b, (((1,), (0,)), ((), ())),
                               precision=jax.lax.Precision.HIGHEST,
                               preferred_element_type=_f32)


def _elu(v):
    return jnp.where(v > 0, v, jnp.exp(jnp.minimum(v, 0.0)) - 1.0)


def _mlp_body(x_ref, w1, b1, g1, e1, w2, b2, g2, e2, w3, b3, o_ref):
    x = x_ref[...]
    h = _elu(_ln(_dot(x, w1[0]) + b1[0], g1[0], e1[0]))
    h = _elu(_ln(_dot(h, w2[0]) + b2[0], g2[0], e2[0]))
    o_ref[...] = _dot(h, w3[0]) + b3[0]


def _stkv(a, b):
    return jnp.stack([a, b]).reshape(2, 1, -1)


def _mlp_all(x_all, w1s, b1s, g1s, e1s, w2s, b2s, g2s, e2s, w3s, b3s):
    nblk = (2 * NPAD) // _MB
    half = nblk // 2
    wmap = lambda i: (i // half, 0, 0)
    vmap = lambda i: (i // half, 0)
    return pl.pallas_call(
        _mlp_body,
        grid=(nblk,),
        in_specs=[
            pl.BlockSpec((_MB, 128), lambda i: (i, 0)),
            pl.BlockSpec((1, 128, 128), wmap),
            pl.BlockSpec((1, 1, 128), wmap),
            pl.BlockSpec((1, 1, 128), wmap),
            pl.BlockSpec((1, 1, 128), wmap),
            pl.BlockSpec((1, 128, 128), wmap),
            pl.BlockSpec((1, 1, 128), wmap),
            pl.BlockSpec((1, 1, 128), wmap),
            pl.BlockSpec((1, 1, 128), wmap),
            pl.BlockSpec((1, 128, D), wmap),
            pl.BlockSpec((1, 1, D), wmap),
        ],
        out_specs=pl.BlockSpec((_MB, D), lambda i: (i, 0)),
        out_shape=jax.ShapeDtypeStruct((2 * NPAD, D), _f32),
    )(x_all, w1s, b1s, g1s, e1s, w2s, b2s, g2s, e2s, w3s, b3s)


_PB = 1024


def _pred_body(zu_ref, zp_ref, zn_ref, w1, b1, w2, b2, w3, o_ref):
    i = pl.program_id(0)
    zu = zu_ref[...]

    def head(z2):
        x = jnp.concatenate([zu, z2], axis=1)
        h = jnp.maximum(_dot(x, w1[...]) + b1[...], 0.0)
        h = jnp.maximum(_dot(h, w2[...]) + b2[...], 0.0)
        return _dot(h, w3[...])

    d = head(zp_ref[...]) - head(zn_ref[...])
    ls = jnp.minimum(d, 0.0) - jnp.log1p(jnp.exp(-jnp.abs(d)))

    @pl.when(i == 0)
    def _():
        o_ref[...] = jnp.zeros((1, 1), _f32)

    o_ref[...] += jnp.sum(ls).reshape(1, 1)


def _pred_loss(zu, zpn, pW1, pb1, pW2, pb2, pW3):
    nblk = B // _PB
    out = pl.pallas_call(
        _pred_body,
        grid=(nblk,),
        in_specs=[
            pl.BlockSpec((_PB, D), lambda i: (i, 0)),
            pl.BlockSpec((_PB, D), lambda i: (i, 0)),
            pl.BlockSpec((_PB, D), lambda i: (i + nblk, 0)),
            pl.BlockSpec((2 * D, 128), lambda i: (0, 0)),
            pl.BlockSpec((1, 128), lambda i: (0, 0)),
            pl.BlockSpec((128, D), lambda i: (0, 0)),
            pl.BlockSpec((1, D), lambda i: (0, 0)),
            pl.BlockSpec((D, 1), lambda i: (0, 0)),
        ],
        out_specs=pl.BlockSpec((1, 1), lambda i: (0, 0)),
        out_shape=jax.ShapeDtypeStruct((1, 1), _f32),
    )(zu, zpn, zpn, pW1, pb1.reshape(1, 128), pW2, pb2.reshape(1, D), pW3)
    return -out[0, 0] / B


def _prep_edges(ei):
    pad = ET - E
    src = jnp.concatenate([ei[0], jnp.zeros((pad,), jnp.int32)])
    dst = jnp.concatenate([ei[1], jnp.full((pad,), N, jnp.int32)])
    return src.reshape(-1, CB), dst.reshape(-1, CB)


def _pad_rows(x):
    return jnp.concatenate([x, jnp.zeros((NPAD - N, x.shape[1]), x.dtype)])


def kernel(x_user, x_item, edge_index_ui, edge_index_iu, user_idx, pos_idx, neg_idx,
           uW1, ub1, ug1, ue1, uW2, ub2, ug2, ue2, uW3, ub3,
           iW1, ib1, ig1, ie1, iW2, ib2, ig2, ie2, iW3, ib3,
           pW1, pb1, pW2, pb2, pW3, pb3):
    x_all = jnp.concatenate([_pad_rows(x_user), _pad_rows(x_item)])
    stk = lambda a, b: jnp.stack([a, b])
    h0 = _mlp_all(x_all,
                  stk(uW1, iW1), _stkv(ub1, ib1), _stkv(ug1, ig1), _stkv(ue1, ie1),
                  stk(uW2, iW2), _stkv(ub2, ib2), _stkv(ug2, ig2), _stkv(ue2, ie2),
                  stk(uW3, iW3), _stkv(ub3, ib3))
    h0u, h0i = h0[:NPAD], h0[NPAD:]

    sui, dui = _prep_edges(edge_index_ui)
    siu, diu = _prep_edges(edge_index_iu)

    reci, recu = _sc_counts(dui, diu)

    hu, hi = h0u, h0i
    for _ in range(K):
        hu, hi = _sc_prop(hu, hi, h0u, h0i, recu, reci, sui, dui, siu, diu)

    zu = _gather_u(hu, user_idx.reshape(-1, CB))
    zpn = _gather_pn(hi, jnp.concatenate([pos_idx, neg_idx]).reshape(-1, CB))

    return _pred_loss(zu, zpn, pW1, pb1, pW2, pb2, pW3)


# speedup vs baseline: 1.1955x; 1.0578x over previous
import functools

import jax
import jax.numpy as jnp
from jax import lax
from jax.experimental import pallas as pl
from jax.experimental.pallas import tpu as pltpu
from jax.experimental.pallas import tpu_sc as plsc

ALPHA = 0.15
K = 10
N = 25000
NPAD = 25088
NROW = NPAD // 16
D = 64
E = 400000
CB = 128
NCHUNK = 196
EPT = NCHUNK * CB
ET = EPT * 16
B = 16384

_f32 = jnp.float32
_mesh = plsc.VectorSubcoreMesh(core_axis_name="c", subcore_axis_name="s")
_sc_params = pltpu.CompilerParams(use_tc_tiling_on_sc=False)


def _fill(ref, value):
    v = jnp.full((16,), value, _f32)
    nrows = ref.shape[0]

    def body(r, carry):
        for c in range(ref.shape[1] // 16):
            ref[r, pl.ds(c * 16, 16)] = v
        return carry

    lax.fori_loop(0, nrows, body, 0)


GRP = 14
NGRP = NCHUNK // GRP
NBUF = 3


@functools.partial(
    pl.kernel,
    out_type=(jax.ShapeDtypeStruct((NPAD, D), _f32),
              jax.ShapeDtypeStruct((NPAD, D), _f32)),
    mesh=_mesh,
    scratch_types=[
        pltpu.VMEM((GRP, CB), jnp.int32),
        pltpu.VMEM((CB, D), _f32),
        pltpu.VMEM((CB, D), _f32),
        pltpu.VMEM((CB, D), _f32),
        pltpu.SemaphoreType.DMA,
        pltpu.SemaphoreType.DMA,
        pltpu.SemaphoreType.DMA,
        pltpu.VMEM_SHARED((NPAD, D), _f32),
    ],
    compiler_params=_sc_params,
)
def _sc_counts(dui, diu, reci_out, recu_out, dgrp, ones, ba, bb,
               ss0, ss1, ss2, acc):
    cid = lax.axis_index("c")
    sid = lax.axis_index("s")
    ssems = (ss0, ss1, ss2)
    base = sid * NROW

    _fill(ba, 0.0)
    _fill(ones, 1.0)
    zd = []
    for k in range(NROW // CB):
        zd.append(pltpu.async_copy(ba, acc.at[pl.ds(base + k * CB, CB)],
                                   ssems[k % 2]))
    zd.append(pltpu.async_copy(ba.at[pl.ds(0, NROW % CB)],
                               acc.at[pl.ds(base + (NROW // CB) * CB, NROW % CB)],
                               ssems[2]))
    for d in zd:
        d.wait()
    plsc.subcore_barrier()

    def count(dm):
        def group(g, carry):
            brow = sid * NCHUNK + g * GRP
            pltpu.sync_copy(dm.at[pl.ds(brow, GRP)], dgrp)
            sd = {}
            for t in range(GRP):
                b = t % NBUF
                if t >= NBUF:
                    sd[t - NBUF].wait()
                sd[t] = pltpu.async_copy(ones, acc.at[dgrp.at[t]],
                                         ssems[b], add=True)
            for t in range(GRP - NBUF, GRP):
                sd[t].wait()
            return carry
        lax.fori_loop(0, NGRP, group, 0)

    @pl.when(cid == 0)
    def _():
        count(dui)

    @pl.when(cid == 1)
    def _():
        count(diu)

    plsc.subcore_barrier()

    def emit(out):
        def sub_block(rb, nr):
            d0 = pltpu.async_copy(acc.at[pl.ds(rb, nr)], ba.at[pl.ds(0, nr)], ss0)
            d0.wait()

            def body(r, c2):
                for c in range(4):
                    s = pl.ds(c * 16, 16)
                    bb[r, s] = (1.0 - ALPHA) / jnp.maximum(ba[r, s], 1.0)
                return c2
            lax.fori_loop(0, nr, body, 0)
            pltpu.sync_copy(bb.at[pl.ds(0, nr)], out.at[pl.ds(rb, nr)])

        def sub(k, carry):
            sub_block(base + k * CB, CB)
            return carry
        lax.fori_loop(0, NROW // CB, sub, 0)
        sub_block(base + (NROW // CB) * CB, NROW % CB)

    @pl.when(cid == 0)
    def _():
        emit(reci_out)

    @pl.when(cid == 1)
    def _():
        emit(recu_out)


@functools.partial(
    pl.kernel,
    out_type=(jax.ShapeDtypeStruct((NPAD, D), _f32),
              jax.ShapeDtypeStruct((NPAD, D), _f32)),
    mesh=_mesh,
    scratch_types=[
        pltpu.VMEM((GRP, CB), jnp.int32),
        pltpu.VMEM((GRP, CB), jnp.int32),
        pltpu.VMEM((CB, D), _f32),
        pltpu.VMEM((CB, D), _f32),
        pltpu.VMEM((CB, D), _f32),
        pltpu.SemaphoreType.DMA,
        pltpu.SemaphoreType.DMA,
        pltpu.SemaphoreType.DMA,
        pltpu.SemaphoreType.DMA,
        pltpu.SemaphoreType.DMA,
        pltpu.SemaphoreType.DMA,
        pltpu.VMEM_SHARED((NPAD, D), _f32),
    ],
    compiler_params=_sc_params,
)
def _sc_prop(hu, hi, h0u, h0i, recu, reci, sui, dui, siu, diu,
             hu_out, hi_out, sgrp, dgrp, rowsA, rowsB, rowsC,
             gs0, gs1, gs2, ss0, ss1, ss2, acc):
    cid = lax.axis_index("c")
    sid = lax.axis_index("s")
    rows = (rowsA, rowsB, rowsC)
    gsems = (gs0, gs1, gs2)
    ssems = (ss0, ss1, ss2)
    base = sid * NROW

    _fill(rowsA, 0.0)
    zd = []
    for k in range(NROW // CB):
        zd.append(pltpu.async_copy(rowsA, acc.at[pl.ds(base + k * CB, CB)],
                                   gsems[k % 2]))
    zd.append(pltpu.async_copy(rowsA.at[pl.ds(0, NROW % CB)],
                               acc.at[pl.ds(base + (NROW // CB) * CB, NROW % CB)],
                               gsems[2]))
    for d in zd:
        d.wait()
    plsc.subcore_barrier()

    def edges(h, sm, dm):
        def group(g, carry):
            brow = sid * NCHUNK + g * GRP
            i0 = pltpu.async_copy(sm.at[pl.ds(brow, GRP)], sgrp, gs0)
            i1 = pltpu.async_copy(dm.at[pl.ds(brow, GRP)], dgrp, gs1)
            i0.wait()
            i1.wait()
            gd, sd = {}, {}
            for t in range(min(NBUF, GRP)):
                gd[t] = pltpu.async_copy(h.at[sgrp.at[t]], rows[t % NBUF],
                                         gsems[t % NBUF])
            for t in range(GRP):
                b = t % NBUF
                gd[t].wait()
                sd[t] = pltpu.async_copy(rows[b], acc.at[dgrp.at[t]],
                                         ssems[b], add=True)
                u = t + NBUF
                if u < GRP:
                    sd[t].wait()
                    gd[u] = pltpu.async_copy(h.at[sgrp.at[u]], rows[b],
                                             gsems[b])
            for t in range(GRP - NBUF, GRP):
                sd[t].wait()
            return carry
        lax.fori_loop(0, NGRP, group, 0)

    @pl.when(cid == 0)
    def _():
        edges(hu, sui, dui)

    @pl.when(cid == 1)
    def _():
        edges(hi, siu, diu)

    plsc.subcore_barrier()

    def blend(rec, h0, hout):
        def sub_block(rb, nr):
            d0 = pltpu.async_copy(acc.at[pl.ds(rb, nr)], rowsA.at[pl.ds(0, nr)], gs0)
            d1 = pltpu.async_copy(rec.at[pl.ds(rb, nr)], rowsB.at[pl.ds(0, nr)], gs1)
            d2 = pltpu.async_copy(h0.at[pl.ds(rb, nr)], rowsC.at[pl.ds(0, nr)], gs2)
            d0.wait()
            d1.wait()
            d2.wait()

            def body(r, c2):
                for c in range(4):
                    s = pl.ds(c * 16, 16)
                    rowsC[r, s] = rowsA[r, s] * rowsB[r, s] + ALPHA * rowsC[r, s]
                return c2
            lax.fori_loop(0, nr, body, 0)
            pltpu.sync_copy(rowsC.at[pl.ds(0, nr)], hout.at[pl.ds(rb, nr)])

        def sub(k, carry):
            sub_block(base + k * CB, CB)
            return carry
        lax.fori_loop(0, NROW // CB, sub, 0)
        sub_block(base + (NROW // CB) * CB, NROW % CB)

    @pl.when(cid == 0)
    def _():
        blend(reci, h0i, hi_out)

    @pl.when(cid == 1)
    def _():
        blend(recu, h0u, hu_out)


def _make_gather(nch, nrows):
    @functools.partial(
        pl.kernel,
        out_type=jax.ShapeDtypeStruct((nrows, D), _f32),
        mesh=_mesh,
        scratch_types=[
            pltpu.VMEM((CB,), jnp.int32),
            pltpu.VMEM((CB, D), _f32),
            pltpu.SemaphoreType.DMA,
        ],
        compiler_params=_sc_params,
    )
    def g(table, idxm, out, sidx, rows, sem):
        cid = lax.axis_index("c")
        sid = lax.axis_index("s")
        wid = sid * 2 + cid

        def body(j, carry):
            row = wid * nch + j
            pltpu.sync_copy(idxm.at[row], sidx)
            pltpu.async_copy(table.at[sidx], rows, sem).wait()
            pltpu.sync_copy(rows, out.at[pl.ds(row * CB, CB)])
            return carry
        lax.fori_loop(0, nch, body, 0)

    return g


_gather_u = _make_gather(4, B)
_gather_pn = _make_gather(8, 2 * B)


_MB = 3136


def _ln(v, g, e):
    mu = jnp.mean(v, axis=-1, keepdims=True)
    var = jnp.mean((v - mu) ** 2, axis=-1, keepdims=True)
    return (v - mu) / jnp.sqrt(var + 1e-5) * g + e


def _dot(a, b):
    return jax.lax.dot_general(a, b, (((1,), (0,)), ((), ())),
                               preferred_element_type=_f32)


def _elu(v):
    return jnp.where(v > 0, v, jnp.exp(jnp.minimum(v, 0.0)) - 1.0)


def _mlp_body(x_ref, w1, b1, g1, e1, w2, b2, g2, e2, w3, b3, o_ref):
    x = x_ref[...]
    h = _elu(_ln(_dot(x, w1[0]) + b1[0], g1[0], e1[0]))
    h = _elu(_ln(_dot(h, w2[0]) + b2[0], g2[0], e2[0]))
    o_ref[...] = _dot(h, w3[0]) + b3[0]


def _stkv(a, b):
    return jnp.stack([a, b]).reshape(2, 1, -1)


def _mlp_all(x_all, w1s, b1s, g1s, e1s, w2s, b2s, g2s, e2s, w3s, b3s):
    nblk = (2 * NPAD) // _MB
    half = nblk // 2
    wmap = lambda i: (i // half, 0, 0)
    vmap = lambda i: (i // half, 0)
    return pl.pallas_call(
        _mlp_body,
        grid=(nblk,),
        in_specs=[
            pl.BlockSpec((_MB, 128), lambda i: (i, 0)),
            pl.BlockSpec((1, 128, 128), wmap),
            pl.BlockSpec((1, 1, 128), wmap),
            pl.BlockSpec((1, 1, 128), wmap),
            pl.BlockSpec((1, 1, 128), wmap),
            pl.BlockSpec((1, 128, 128), wmap),
            pl.BlockSpec((1, 1, 128), wmap),
            pl.BlockSpec((1, 1, 128), wmap),
            pl.BlockSpec((1, 1, 128), wmap),
            pl.BlockSpec((1, 128, D), wmap),
            pl.BlockSpec((1, 1, D), wmap),
        ],
        out_specs=pl.BlockSpec((_MB, D), lambda i: (i, 0)),
        out_shape=jax.ShapeDtypeStruct((2 * NPAD, D), _f32),
    )(x_all, w1s, b1s, g1s, e1s, w2s, b2s, g2s, e2s, w3s, b3s)


_PB = 1024


def _pred_body(zu_ref, zp_ref, zn_ref, w1, b1, w2, b2, w3, o_ref):
    i = pl.program_id(0)
    zu = zu_ref[...]

    def head(z2):
        x = jnp.concatenate([zu, z2], axis=1)
        h = jnp.maximum(_dot(x, w1[...]) + b1[...], 0.0)
        h = jnp.maximum(_dot(h, w2[...]) + b2[...], 0.0)
        return _dot(h, w3[...])

    d = head(zp_ref[...]) - head(zn_ref[...])
    ls = jnp.minimum(d, 0.0) - jnp.log1p(jnp.exp(-jnp.abs(d)))

    @pl.when(i == 0)
    def _():
        o_ref[...] = jnp.zeros((1, 1), _f32)

    o_ref[...] += jnp.sum(ls).reshape(1, 1)


def _pred_loss(zu, zpn, pW1, pb1, pW2, pb2, pW3):
    nblk = B // _PB
    out = pl.pallas_call(
        _pred_body,
        grid=(nblk,),
        in_specs=[
            pl.BlockSpec((_PB, D), lambda i: (i, 0)),
            pl.BlockSpec((_PB, D), lambda i: (i, 0)),
            pl.BlockSpec((_PB, D), lambda i: (i + nblk, 0)),
            pl.BlockSpec((2 * D, 128), lambda i: (0, 0)),
            pl.BlockSpec((1, 128), lambda i: (0, 0)),
            pl.BlockSpec((128, D), lambda i: (0, 0)),
            pl.BlockSpec((1, D), lambda i: (0, 0)),
            pl.BlockSpec((D, 1), lambda i: (0, 0)),
        ],
        out_specs=pl.BlockSpec((1, 1), lambda i: (0, 0)),
        out_shape=jax.ShapeDtypeStruct((1, 1), _f32),
    )(zu, zpn, zpn, pW1, pb1.reshape(1, 128), pW2, pb2.reshape(1, D), pW3)
    return -out[0, 0] / B


def _prep_edges(ei):
    pad = ET - E
    src = jnp.concatenate([ei[0], jnp.zeros((pad,), jnp.int32)])
    dst = jnp.concatenate([ei[1], jnp.full((pad,), N, jnp.int32)])
    return src.reshape(-1, CB), dst.reshape(-1, CB)


def _pad_rows(x):
    return jnp.concatenate([x, jnp.zeros((NPAD - N, x.shape[1]), x.dtype)])


def kernel(x_user, x_item, edge_index_ui, edge_index_iu, user_idx, pos_idx, neg_idx,
           uW1, ub1, ug1, ue1, uW2, ub2, ug2, ue2, uW3, ub3,
           iW1, ib1, ig1, ie1, iW2, ib2, ig2, ie2, iW3, ib3,
           pW1, pb1, pW2, pb2, pW3, pb3):
    x_all = jnp.concatenate([_pad_rows(x_user), _pad_rows(x_item)])
    stk = lambda a, b: jnp.stack([a, b])
    h0 = _mlp_all(x_all,
                  stk(uW1, iW1), _stkv(ub1, ib1), _stkv(ug1, ig1), _stkv(ue1, ie1),
                  stk(uW2, iW2), _stkv(ub2, ib2), _stkv(ug2, ig2), _stkv(ue2, ie2),
                  stk(uW3, iW3), _stkv(ub3, ib3))
    h0u, h0i = h0[:NPAD], h0[NPAD:]

    sui, dui = _prep_edges(edge_index_ui)
    siu, diu = _prep_edges(edge_index_iu)

    reci, recu = _sc_counts(dui, diu)

    hu, hi = h0u, h0i
    for _ in range(K):
        hu, hi = _sc_prop(hu, hi, h0u, h0i, recu, reci, sui, dui, siu, diu)

    zu = _gather_u(hu, user_idx.reshape(-1, CB))
    zpn = _gather_pn(hi, jnp.concatenate([pos_idx, neg_idx]).reshape(-1, CB))

    return _pred_loss(zu, zpn, pW1, pb1, pW2, pb2, pW3)
